# baseline (speedup 1.0000x reference)
import functools

import jax
import jax.numpy as jnp
from jax import lax
from jax.experimental import pallas as pl
from jax.experimental.pallas import tpu as pltpu
from jax.experimental.pallas import tpu_sc as plsc

VOCAB = 100000
D_MODEL = 1024
PATCH = 16
NPATCH = 14
NTOK = NPATCH * NPATCH
PDIM = 3 * PATCH * PATCH
NPIECE = 48
PCW = NPATCH * PATCH


def _make_sc_gather(Bt, T, D, chunk):
    info = plsc.get_sparse_core_info()
    nw = info.num_cores * info.num_subcores
    b_per_w = Bt * T // nw
    n_ch = b_per_w // chunk
    w_per_row = T // b_per_w
    mesh = plsc.VectorSubcoreMesh(core_axis_name="c", subcore_axis_name="s")

    nbuf = 4

    @functools.partial(
        pl.kernel,
        mesh=mesh,
        out_type=jax.ShapeDtypeStruct((Bt, T, D), jnp.float32),
        scratch_types=[
            pltpu.VMEM((n_ch, chunk), jnp.int32),
            pltpu.VMEM((nbuf, chunk, D), jnp.float32),
            pltpu.SemaphoreType.DMA,
            pltpu.SemaphoreType.DMA,
        ],
    )
    def gather_k(table_hbm, idx_hbm, out_hbm, idx_v, rows_v, sem_g, sem_w):
        wid = lax.axis_index("s") * info.num_cores + lax.axis_index("c")
        bt = wid // w_per_row
        t0 = (wid % w_per_row) * b_per_w
        pltpu.sync_copy(idx_hbm.at[wid], idx_v)

        def gath(j, b):
            return pltpu.async_copy(
                table_hbm.at[idx_v.at[j]], rows_v.at[b], sem_g)

        rh = [None] * nbuf
        wh = [None] * nbuf
        for j in range(min(nbuf, n_ch)):
            rh[j] = gath(j, j)
        for j in range(n_ch):
            b = j % nbuf
            rh[b].wait()
            wh[b] = pltpu.async_copy(
                rows_v.at[b], out_hbm.at[bt, pl.ds(t0 + j * chunk, chunk)],
                sem_w)
            if j + nbuf < n_ch:
                wh[b].wait()
                rh[b] = gath(j + nbuf, b)
        for b in range(nbuf):
            if wh[b] is not None:
                wh[b].wait()

    return gather_k, nw, n_ch


NGRP = 6


def _make_sc_patchify(bs):
    info = plsc.get_sparse_core_info()
    mesh = plsc.VectorSubcoreMesh(core_axis_name="c", subcore_axis_name="s")

    @functools.partial(
        pl.kernel,
        mesh=mesh,
        out_type=jax.ShapeDtypeStruct((bs, NGRP, NPATCH, NPATCH, 128),
                                      jnp.float32),
        scratch_types=[
            pltpu.VMEM((2, NPATCH, 8, PCW), jnp.float32),
            pltpu.VMEM((2, NPATCH, NPATCH, 128), jnp.float32),
            pltpu.SemaphoreType.DMA,
            pltpu.SemaphoreType.DMA,
        ],
    )
    def patchify_k(img_hbm, out_hbm, slab, grp, sem_g, sem_w):
        wid = lax.axis_index("s") * info.num_cores + lax.axis_index("c")
        s = wid // 2
        g0 = (wid % 2) * 3

        def read(k, b):
            g = g0 + k
            c = g // 2
            ph0 = pl.multiple_of((g % 2) * 8, 8)
            return pltpu.async_copy(
                img_hbm.at[s, c, :, pl.ds(ph0, 8), :], slab.at[b], sem_g)

        rh = [None, None]
        wh = [None, None]
        rh[0] = read(0, 0)
        for k in range(3):
            b = k % 2
            rh[b].wait()
            if k + 1 < 3:
                rh[(k + 1) % 2] = read(k + 1, (k + 1) % 2)
            if wh[b] is not None:
                wh[b].wait()

            def body(pr, _):
                for m in range(8):
                    for pc in range(NPATCH):
                        v = slab[b, pr, m, pl.ds(pc * PATCH, PATCH)]
                        grp[b, pr, pc, pl.ds(m * PATCH, PATCH)] = v
                return 0

            lax.fori_loop(0, NPATCH, body, 0)
            wh[b] = pltpu.async_copy(grp.at[b], out_hbm.at[s, g0 + k], sem_w)
        for b in range(2):
            if wh[b] is not None:
                wh[b].wait()

    return patchify_k


def _img_body(x_ref, w_ref, b_ref, o_ref):
    total = jnp.float32(0.0)
    acc = jnp.broadcast_to(b_ref[...], (NTOK, D_MODEL))
    for g in range(NGRP):
        xg = x_ref[0, g].reshape(NTOK, 128)
        total += jnp.sum(xg)
        acc = acc + jnp.dot(xg, w_ref[g], preferred_element_type=jnp.float32)
    o_ref[0] = jnp.where(total != 0.0, acc, jnp.zeros((), jnp.float32))


def _img_embed(groups, W_img, b_img):
    bs = groups.shape[0]
    w3 = W_img.reshape(NGRP, 128, D_MODEL)
    return pl.pallas_call(
        _img_body,
        grid=(bs,),
        in_specs=[
            pl.BlockSpec((1, NGRP, NPATCH, NPATCH, 128),
                         lambda i: (i, 0, 0, 0, 0)),
            pl.BlockSpec((NGRP, 128, D_MODEL), lambda i: (0, 0, 0)),
            pl.BlockSpec((1, D_MODEL), lambda i: (0, 0)),
        ],
        out_specs=pl.BlockSpec((1, NTOK, D_MODEL), lambda i: (i, 0, 0)),
        out_shape=jax.ShapeDtypeStruct((bs, NTOK, D_MODEL), jnp.float32),
    )(groups, w3, b_img)


def kernel(text_input_ids, image_raw_inputs, text_table, W_img, b_img):
    Bt, T = text_input_ids.shape
    chunk = 16
    gather_k, nw, n_ch = _make_sc_gather(Bt, T, D_MODEL, chunk)
    ids = text_input_ids.reshape(nw, n_ch, chunk).astype(jnp.int32)
    text_out = gather_k(text_table, ids)

    B, N, C, H, W = image_raw_inputs.shape
    bs = B * N
    img5 = image_raw_inputs.reshape(bs, C, NPATCH, PATCH, PCW)
    patchify_k = _make_sc_patchify(bs)
    groups = patchify_k(img5)
    img = _img_embed(groups, W_img, b_img.reshape(1, D_MODEL))
    image_logits = img.reshape(B, N, NTOK, D_MODEL)

    return (text_out, image_logits)


# speedup vs baseline: 1.0063x; 1.0063x over previous
import functools

import jax
import jax.numpy as jnp
from jax import lax
from jax.experimental import pallas as pl
from jax.experimental.pallas import tpu as pltpu
from jax.experimental.pallas import tpu_sc as plsc

VOCAB = 100000
D_MODEL = 1024
PATCH = 16
NPATCH = 14
NTOK = NPATCH * NPATCH
PDIM = 3 * PATCH * PATCH
NPIECE = 48
PCW = NPATCH * PATCH


def _make_sc_gather(Bt, T, D, chunk):
    info = plsc.get_sparse_core_info()
    nw = info.num_cores * info.num_subcores
    b_per_w = Bt * T // nw
    n_ch = b_per_w // chunk
    w_per_row = T // b_per_w
    mesh = plsc.VectorSubcoreMesh(core_axis_name="c", subcore_axis_name="s")

    nbuf = 3

    @functools.partial(
        pl.kernel,
        mesh=mesh,
        out_type=jax.ShapeDtypeStruct((Bt, T, D), jnp.float32),
        scratch_types=[
            pltpu.VMEM((n_ch, chunk), jnp.int32),
            pltpu.VMEM((nbuf, chunk, D), jnp.float32),
            pltpu.SemaphoreType.DMA,
            pltpu.SemaphoreType.DMA,
        ],
    )
    def gather_k(table_hbm, idx_hbm, out_hbm, idx_v, rows_v, sem_g, sem_w):
        wid = lax.axis_index("s") * info.num_cores + lax.axis_index("c")
        bt = wid // w_per_row
        t0 = (wid % w_per_row) * b_per_w
        pltpu.sync_copy(idx_hbm.at[wid], idx_v)

        def gath(j, b):
            return pltpu.async_copy(
                table_hbm.at[idx_v.at[j]], rows_v.at[b], sem_g)

        rh = [None] * nbuf
        wh = [None] * nbuf
        for j in range(min(nbuf, n_ch)):
            rh[j] = gath(j, j)
        for j in range(n_ch):
            b = j % nbuf
            rh[b].wait()
            wh[b] = pltpu.async_copy(
                rows_v.at[b], out_hbm.at[bt, pl.ds(t0 + j * chunk, chunk)],
                sem_w)
            if j + nbuf < n_ch:
                wh[b].wait()
                rh[b] = gath(j + nbuf, b)
        for b in range(nbuf):
            if wh[b] is not None:
                wh[b].wait()

    return gather_k, nw, n_ch


NGRP = 6


def _make_sc_patchify(bs):
    info = plsc.get_sparse_core_info()
    mesh = plsc.VectorSubcoreMesh(core_axis_name="c", subcore_axis_name="s")

    @functools.partial(
        pl.kernel,
        mesh=mesh,
        out_type=jax.ShapeDtypeStruct((bs, NGRP, NPATCH, NPATCH, 128),
                                      jnp.float32),
        scratch_types=[
            pltpu.VMEM((2, NPATCH, 8, PCW), jnp.float32),
            pltpu.VMEM((2, NPATCH, NPATCH, 128), jnp.float32),
            pltpu.SemaphoreType.DMA,
            pltpu.SemaphoreType.DMA,
        ],
    )
    def patchify_k(img_hbm, out_hbm, slab, grp, sem_g, sem_w):
        wid = lax.axis_index("s") * info.num_cores + lax.axis_index("c")
        s = wid // 2
        g0 = (wid % 2) * 3

        def read(k, b):
            g = g0 + k
            c = g // 2
            ph0 = pl.multiple_of((g % 2) * 8, 8)
            return pltpu.async_copy(
                img_hbm.at[s, c, :, pl.ds(ph0, 8), :], slab.at[b], sem_g)

        rh = [None, None]
        wh = [None, None]
        rh[0] = read(0, 0)
        for k in range(3):
            b = k % 2
            rh[b].wait()
            if k + 1 < 3:
                rh[(k + 1) % 2] = read(k + 1, (k + 1) % 2)
            if wh[b] is not None:
                wh[b].wait()

            def body(pr, _):
                for m in range(8):
                    for pc in range(NPATCH):
                        v = slab[b, pr, m, pl.ds(pc * PATCH, PATCH)]
                        grp[b, pr, pc, pl.ds(m * PATCH, PATCH)] = v
                return 0

            lax.fori_loop(0, NPATCH, body, 0)
            wh[b] = pltpu.async_copy(grp.at[b], out_hbm.at[s, g0 + k], sem_w)
        for b in range(2):
            if wh[b] is not None:
                wh[b].wait()

    return patchify_k


def _img_body(x_ref, w_ref, b_ref, o_ref):
    total = jnp.float32(0.0)
    acc = jnp.broadcast_to(b_ref[...], (NTOK, D_MODEL))
    for g in range(NGRP):
        xg = x_ref[0, g].reshape(NTOK, 128)
        total += jnp.sum(xg)
        acc = acc + jnp.dot(xg, w_ref[g], preferred_element_type=jnp.float32)
    o_ref[0] = jnp.where(total != 0.0, acc, jnp.zeros((), jnp.float32))


def _img_embed(groups, W_img, b_img):
    bs = groups.shape[0]
    w3 = W_img.reshape(NGRP, 128, D_MODEL)
    return pl.pallas_call(
        _img_body,
        grid=(bs,),
        in_specs=[
            pl.BlockSpec((1, NGRP, NPATCH, NPATCH, 128),
                         lambda i: (i, 0, 0, 0, 0)),
            pl.BlockSpec((NGRP, 128, D_MODEL), lambda i: (0, 0, 0)),
            pl.BlockSpec((1, D_MODEL), lambda i: (0, 0)),
        ],
        out_specs=pl.BlockSpec((1, NTOK, D_MODEL), lambda i: (i, 0, 0)),
        out_shape=jax.ShapeDtypeStruct((bs, NTOK, D_MODEL), jnp.float32),
    )(groups, w3, b_img)


def kernel(text_input_ids, image_raw_inputs, text_table, W_img, b_img):
    Bt, T = text_input_ids.shape
    chunk = 32
    gather_k, nw, n_ch = _make_sc_gather(Bt, T, D_MODEL, chunk)
    ids = text_input_ids.reshape(nw, n_ch, chunk).astype(jnp.int32)
    text_out = gather_k(text_table, ids)

    B, N, C, H, W = image_raw_inputs.shape
    bs = B * N
    img5 = image_raw_inputs.reshape(bs, C, NPATCH, PATCH, PCW)
    patchify_k = _make_sc_patchify(bs)
    groups = patchify_k(img5)
    img = _img_embed(groups, W_img, b_img.reshape(1, D_MODEL))
    image_logits = img.reshape(B, N, NTOK, D_MODEL)

    return (text_out, image_logits)
